# baseline (speedup 1.0000x reference)
import dataclasses
import functools

import jax
import jax.numpy as jnp
from jax import lax
from jax.experimental import pallas as pl
from jax.experimental.pallas import tpu as pltpu
from jax.experimental.pallas import tpu_sc as plsc

NC = 2
NS = 16
NW = NC * NS
CH = 64


def _sc_segment_sum(src, dst, x2d, zx, n_pad):
    e = src.shape[0]
    d = x2d.shape[1]
    epw = e // NW
    cpw = epw // CH
    mesh = plsc.VectorSubcoreMesh(core_axis_name="c", subcore_axis_name="s", num_cores=NC)
    rows_per_sub = n_pad // NS

    cp = pltpu.CompilerParams()
    if "needs_layout_passes" in pltpu.CompilerParams.__dataclass_fields__:
        cp = dataclasses.replace(cp, needs_layout_passes=False)

    @functools.partial(
        pl.kernel,
        compiler_params=cp,
        out_type=[
            jax.ShapeDtypeStruct((NC, n_pad, d), jnp.float32),
            jax.ShapeDtypeStruct((NW * n_pad,), jnp.float32),
        ],
        mesh=mesh,
        scratch_types=(
            [pltpu.VMEM((CH,), jnp.int32)] * 4
            + [pltpu.VMEM((CH,), jnp.int32)] * 4
            + [pltpu.VMEM((CH, d), jnp.float32)] * 4
            + [
                pltpu.VMEM((n_pad,), jnp.float32),
                pltpu.VMEM_SHARED((n_pad, d), jnp.float32),
            ]
            + [pltpu.SemaphoreType.DMA] * 16
        ),
    )
    def seg_sum(src_hbm, dst_hbm, x_hbm, zx_hbm, accx_hbm, cpart_hbm,
                *refs):
        sidx_b = refs[0:4]
        didx_b = refs[4:8]
        rows_b = refs[8:12]
        cnt_tile, accx_sh = refs[12:14]
        gsem = refs[14:18]
        issem = refs[18:22]
        idsem = refs[22:26]
        ssem = refs[26:30]

        c = lax.axis_index("c")
        s = lax.axis_index("s")
        rbase = s * rows_per_sub
        wbase = (c * NS + s) * epw

        z16 = jnp.zeros((16,), jnp.float32)
        one16 = jnp.full((16,), 1.0, jnp.float32)

        def a_sidx(base, u):
            pltpu.make_async_copy(
                src_hbm.at[pl.ds(base, CH)], sidx_b[u], issem[u]).start()

        def a_didx(base, u):
            pltpu.make_async_copy(
                dst_hbm.at[pl.ds(base, CH)], didx_b[u], idsem[u]).start()

        def b_gather(base, u, wait_scatter):
            if wait_scatter:
                pltpu.make_async_copy(
                    rows_b[u], accx_sh.at[didx_b[u]], ssem[u]).wait()
            pltpu.make_async_copy(
                src_hbm.at[pl.ds(base, CH)], sidx_b[u], issem[u]).wait()
            pltpu.make_async_copy(
                x_hbm.at[sidx_b[u]], rows_b[u], gsem[u]).start()

        def c_consume(base, u):
            pltpu.make_async_copy(
                dst_hbm.at[pl.ds(base, CH)], didx_b[u], idsem[u]).wait()
            pltpu.make_async_copy(
                x_hbm.at[sidx_b[u]], rows_b[u], gsem[u]).wait()
            pltpu.make_async_copy(
                rows_b[u], accx_sh.at[didx_b[u]], ssem[u]).start(add=True)

            @pl.loop(0, CH, step=16)
            def _(j):
                idx16 = didx_b[u].at[pl.ds(j, 16)][...]
                plsc.addupdate_scatter(cnt_tile, [idx16], one16)

        for k in range(4):
            a_sidx(wbase + k * CH, k % 4)
        for k in range(3):
            a_didx(wbase + k * CH, k % 4)

        @pl.loop(0, n_pad, step=16)
        def _(r):
            cnt_tile.at[pl.ds(r, 16)][...] = z16

        pltpu.sync_copy(zx_hbm, accx_sh.at[pl.ds(rbase, rows_per_sub)])
        plsc.subcore_barrier()

        for k in range(3):
            b_gather(wbase + k * CH, k % 4, wait_scatter=False)

        def slot(k_dyn, u, wait_b):
            c_consume(k_dyn, u)
            a_sidx(k_dyn + 4 * CH, u)
            b_gather(k_dyn + 3 * CH, (u + 3) % 4, wait_scatter=wait_b)
            a_didx(k_dyn + 3 * CH, (u + 3) % 4)

        for u in range(4):
            slot(wbase + u * CH, u, wait_b=(u >= 1))

        @pl.loop(4, cpw - 4, step=4)
        def _(t):
            base = wbase + t * CH
            for u in range(4):
                slot(base + u * CH, u, wait_b=True)

        for u in range(4):
            k = cpw - 4 + u
            c_consume(wbase + k * CH, u)
            if k + 3 < cpw:
                b_gather(wbase + (k + 3) * CH, (u + 3) % 4, wait_scatter=True)
                a_didx(wbase + (k + 3) * CH, (u + 3) % 4)
        for u in range(4):
            pltpu.make_async_copy(
                rows_b[u], accx_sh.at[didx_b[u]], ssem[u]).wait()
        plsc.subcore_barrier()

        pltpu.sync_copy(accx_sh.at[pl.ds(rbase, rows_per_sub)],
                        accx_hbm.at[c, pl.ds(rbase, rows_per_sub)])

        pltpu.sync_copy(cnt_tile,
                        cpart_hbm.at[pl.ds((c * NS + s) * n_pad, n_pad)])

    return seg_sum(src, dst, x2d, zx)


def _tc_self(x2d, wsT, bs, n, d, rows):
    def body(x_ref, wsT_ref, bs_ref, o_ref):
        o_ref[...] = jnp.dot(x_ref[...], wsT_ref[...],
                             preferred_element_type=jnp.float32) + bs_ref[...]

    return pl.pallas_call(
        body,
        grid=(n // rows,),
        in_specs=[
            pl.BlockSpec((rows, d), lambda i: (i, 0)),
            pl.BlockSpec((d, d), lambda i: (0, 0)),
            pl.BlockSpec((1, d), lambda i: (0, 0)),
        ],
        out_specs=pl.BlockSpec((rows, d), lambda i: (i, 0)),
        out_shape=jax.ShapeDtypeStruct((n, d), jnp.float32),
    )(x2d, wsT, bs)


def _tc_finish(self_out, accx, cnt3, degf, wnT, bn, n, n_pad, d, rows):
    grid = (n // rows,)

    def body(s_ref, a_ref, c_ref, deg_ref, wnT_ref, bn_ref, o_ref):
        acc = a_ref[0] + a_ref[1]
        cnt = jnp.sum(c_ref[...], axis=0)
        denom = jnp.maximum(deg_ref[...], 1.0)
        inv = 1.0 / denom
        neigh = jnp.dot(acc, wnT_ref[...], preferred_element_type=jnp.float32)
        out = s_ref[...] + (neigh + cnt * bn_ref[...]) * inv
        o_ref[...] = jnp.where(out >= 0, out, 0.1 * out)

    return pl.pallas_call(
        body,
        grid=grid,
        in_specs=[
            pl.BlockSpec((rows, d), lambda i: (i, 0)),
            pl.BlockSpec((NC, rows, d), lambda i: (0, i, 0)),
            pl.BlockSpec((NW, rows, 1), lambda i: (0, i, 0)),
            pl.BlockSpec((rows, 1), lambda i: (i, 0)),
            pl.BlockSpec((d, d), lambda i: (0, 0)),
            pl.BlockSpec((1, d), lambda i: (0, 0)),
        ],
        out_specs=pl.BlockSpec((rows, d), lambda i: (i, 0)),
        out_shape=jax.ShapeDtypeStruct((n, d), jnp.float32),
    )(self_out, accx, cnt3, degf, wnT, bn)


def kernel(x, edge_index, deg, W_self, b_self, W_neigh, b_neigh):
    b, n, d = x.shape
    e = edge_index.shape[1]
    assert b == 1

    src = edge_index[0].astype(jnp.int32)
    dst = edge_index[1].astype(jnp.int32)
    x2d = x.reshape(n, d).astype(jnp.float32)

    n_pad = ((n + 2047) // 2048) * 2048
    assert n_pad >= n + 64
    rows_per_sub = n_pad // NS
    zx = jnp.zeros((rows_per_sub, d), jnp.float32)

    quantum = NW * CH * 4
    e_pad = ((e + quantum - 1) // quantum) * quantum
    e_pad = max(e_pad, 2 * quantum)
    if e_pad != e:
        pad_i = jnp.arange(e_pad - e, dtype=jnp.int32) % 64
        src = jnp.concatenate([src, pad_i])
        dst = jnp.concatenate([dst, n + pad_i])

    accx, cpart = _sc_segment_sum(src, dst, x2d, zx, n_pad)

    self_out = _tc_self(x2d, W_self.T.astype(jnp.float32),
                        b_self.reshape(1, d).astype(jnp.float32), n, d, 2000)

    degf = deg.astype(jnp.float32).reshape(n, 1)
    out2d = _tc_finish(
        self_out, accx, cpart.reshape(NW, n_pad, 1), degf,
        W_neigh.T.astype(jnp.float32),
        b_neigh.reshape(1, d).astype(jnp.float32),
        n, n_pad, d, rows=1000,
    )
    return out2d.reshape(b, n, d).astype(x.dtype)


# speedup vs baseline: 2.0164x; 2.0164x over previous
import dataclasses
import functools

import jax
import jax.numpy as jnp
from jax import lax
from jax.experimental import pallas as pl
from jax.experimental.pallas import tpu as pltpu
from jax.experimental.pallas import tpu_sc as plsc

NC = 2
NS = 16
NW = NC * NS
CH = 64


def _sc_segment_sum(src, dst, x2d, zx, n_pad):
    e = src.shape[0]
    d = x2d.shape[1]
    epw = e // NW
    cpw = epw // CH
    mesh = plsc.VectorSubcoreMesh(core_axis_name="c", subcore_axis_name="s", num_cores=NC)
    rows_per_sub = n_pad // NS

    cp = pltpu.CompilerParams()
    if "needs_layout_passes" in pltpu.CompilerParams.__dataclass_fields__:
        cp = dataclasses.replace(cp, needs_layout_passes=False)

    @functools.partial(
        pl.kernel,
        compiler_params=cp,
        out_type=[
            jax.ShapeDtypeStruct((NC, n_pad, d), jnp.float32),
            jax.ShapeDtypeStruct((NC * n_pad,), jnp.float32),
            jax.ShapeDtypeStruct((NW * n_pad,), jnp.float32),
        ],
        mesh=mesh,
        scratch_types=(
            [pltpu.VMEM((CH,), jnp.int32)] * 4
            + [pltpu.VMEM((CH,), jnp.int32)] * 4
            + [pltpu.VMEM((CH, d), jnp.float32)] * 4
            + [
                pltpu.VMEM((n_pad,), jnp.float32),
                pltpu.VMEM((rows_per_sub,), jnp.float32),
                pltpu.VMEM((rows_per_sub,), jnp.float32),
                pltpu.VMEM_SHARED((n_pad, d), jnp.float32),
            ]
            + [pltpu.SemaphoreType.DMA] * 16
        ),
    )
    def seg_sum(src_hbm, dst_hbm, x_hbm, zx_hbm, accx_hbm, cnt_hbm, cpart_hbm,
                *refs):
        sidx_b = refs[0:4]
        didx_b = refs[4:8]
        rows_b = refs[8:12]
        cnt_tile, red_v, rin_v, accx_sh = refs[12:16]
        gsem = refs[16:20]
        issem = refs[20:24]
        idsem = refs[24:28]
        ssem = refs[28:32]

        c = lax.axis_index("c")
        s = lax.axis_index("s")
        rbase = s * rows_per_sub
        wbase = (c * NS + s) * epw

        z16 = jnp.zeros((16,), jnp.float32)
        one16 = jnp.full((16,), 1.0, jnp.float32)

        def a_sidx(base, u):
            pltpu.make_async_copy(
                src_hbm.at[pl.ds(base, CH)], sidx_b[u], issem[u]).start()

        def a_didx(base, u):
            pltpu.make_async_copy(
                dst_hbm.at[pl.ds(base, CH)], didx_b[u], idsem[u]).start()

        def b_gather(base, u, wait_scatter):
            if wait_scatter:
                pltpu.make_async_copy(
                    rows_b[u], accx_sh.at[didx_b[u]], ssem[u]).wait()
            pltpu.make_async_copy(
                src_hbm.at[pl.ds(base, CH)], sidx_b[u], issem[u]).wait()
            pltpu.make_async_copy(
                x_hbm.at[sidx_b[u]], rows_b[u], gsem[u]).start()

        def c_consume(base, u):
            pltpu.make_async_copy(
                dst_hbm.at[pl.ds(base, CH)], didx_b[u], idsem[u]).wait()
            pltpu.make_async_copy(
                x_hbm.at[sidx_b[u]], rows_b[u], gsem[u]).wait()
            pltpu.make_async_copy(
                rows_b[u], accx_sh.at[didx_b[u]], ssem[u]).start(add=True)

            @pl.loop(0, CH, step=16)
            def _(j):
                idx16 = didx_b[u].at[pl.ds(j, 16)][...]
                plsc.addupdate_scatter(cnt_tile, [idx16], one16)

        for k in range(4):
            a_sidx(wbase + k * CH, k % 4)
        for k in range(3):
            a_didx(wbase + k * CH, k % 4)

        @pl.loop(0, n_pad, step=16)
        def _(r):
            cnt_tile.at[pl.ds(r, 16)][...] = z16

        pltpu.sync_copy(zx_hbm, accx_sh.at[pl.ds(rbase, rows_per_sub)])
        plsc.subcore_barrier()

        for k in range(3):
            b_gather(wbase + k * CH, k % 4, wait_scatter=False)

        def slot(k_dyn, u, wait_b):
            c_consume(k_dyn, u)
            a_sidx(k_dyn + 4 * CH, u)
            b_gather(k_dyn + 3 * CH, (u + 3) % 4, wait_scatter=wait_b)
            a_didx(k_dyn + 3 * CH, (u + 3) % 4)

        for u in range(4):
            slot(wbase + u * CH, u, wait_b=(u >= 1))

        @pl.loop(4, cpw - 4, step=4)
        def _(t):
            base = wbase + t * CH
            for u in range(4):
                slot(base + u * CH, u, wait_b=True)

        for u in range(4):
            k = cpw - 4 + u
            c_consume(wbase + k * CH, u)
            if k + 3 < cpw:
                b_gather(wbase + (k + 3) * CH, (u + 3) % 4, wait_scatter=True)
                a_didx(wbase + (k + 3) * CH, (u + 3) % 4)
        for u in range(4):
            pltpu.make_async_copy(
                rows_b[u], accx_sh.at[didx_b[u]], ssem[u]).wait()
        plsc.subcore_barrier()

        pltpu.sync_copy(accx_sh.at[pl.ds(rbase, rows_per_sub)],
                        accx_hbm.at[c, pl.ds(rbase, rows_per_sub)])

        pltpu.sync_copy(cnt_tile,
                        cpart_hbm.at[pl.ds((c * NS + s) * n_pad, n_pad)])
        plsc.subcore_barrier()

        @pl.loop(0, rows_per_sub, step=16)
        def _(r):
            red_v.at[pl.ds(r, 16)][...] = z16

        for k in range(NS):
            pltpu.sync_copy(
                cpart_hbm.at[pl.ds((c * NS + k) * n_pad + rbase, rows_per_sub)],
                rin_v)

            @pl.loop(0, rows_per_sub, step=16)
            def _(r):
                red_v.at[pl.ds(r, 16)][...] = (
                    red_v.at[pl.ds(r, 16)][...] + rin_v.at[pl.ds(r, 16)][...])

        pltpu.sync_copy(red_v,
                        cnt_hbm.at[pl.ds(c * n_pad + rbase, rows_per_sub)])

    return seg_sum(src, dst, x2d, zx)


def _tc_self(x2d, wsT, bs, n, d, rows):
    def body(x_ref, wsT_ref, bs_ref, o_ref):
        o_ref[...] = jnp.dot(x_ref[...], wsT_ref[...],
                             preferred_element_type=jnp.float32) + bs_ref[...]

    return pl.pallas_call(
        body,
        grid=(n // rows,),
        in_specs=[
            pl.BlockSpec((rows, d), lambda i: (i, 0)),
            pl.BlockSpec((d, d), lambda i: (0, 0)),
            pl.BlockSpec((1, d), lambda i: (0, 0)),
        ],
        out_specs=pl.BlockSpec((rows, d), lambda i: (i, 0)),
        out_shape=jax.ShapeDtypeStruct((n, d), jnp.float32),
    )(x2d, wsT, bs)


def _tc_finish(self_out, accx, cnt3, degf, wnT, bn, n, n_pad, d, rows):
    grid = (n // rows,)

    def body(s_ref, a_ref, c_ref, deg_ref, wnT_ref, bn_ref, o_ref):
        acc = a_ref[0] + a_ref[1]
        cnt = c_ref[0] + c_ref[1]
        denom = jnp.maximum(deg_ref[...], 1.0)
        inv = 1.0 / denom
        neigh = jnp.dot(acc, wnT_ref[...], preferred_element_type=jnp.float32)
        out = s_ref[...] + (neigh + cnt * bn_ref[...]) * inv
        o_ref[...] = jnp.where(out >= 0, out, 0.1 * out)

    return pl.pallas_call(
        body,
        grid=grid,
        in_specs=[
            pl.BlockSpec((rows, d), lambda i: (i, 0)),
            pl.BlockSpec((NC, rows, d), lambda i: (0, i, 0)),
            pl.BlockSpec((NC, rows, 1), lambda i: (0, i, 0)),
            pl.BlockSpec((rows, 1), lambda i: (i, 0)),
            pl.BlockSpec((d, d), lambda i: (0, 0)),
            pl.BlockSpec((1, d), lambda i: (0, 0)),
        ],
        out_specs=pl.BlockSpec((rows, d), lambda i: (i, 0)),
        out_shape=jax.ShapeDtypeStruct((n, d), jnp.float32),
    )(self_out, accx, cnt3, degf, wnT, bn)


def kernel(x, edge_index, deg, W_self, b_self, W_neigh, b_neigh):
    b, n, d = x.shape
    e = edge_index.shape[1]
    assert b == 1

    src = edge_index[0].astype(jnp.int32)
    dst = edge_index[1].astype(jnp.int32)
    x2d = x.reshape(n, d).astype(jnp.float32)

    n_pad = ((n + 2047) // 2048) * 2048
    assert n_pad >= n + 64
    rows_per_sub = n_pad // NS
    zx = jnp.zeros((rows_per_sub, d), jnp.float32)

    quantum = NW * CH * 4
    e_pad = ((e + quantum - 1) // quantum) * quantum
    e_pad = max(e_pad, 2 * quantum)
    if e_pad != e:
        pad_i = jnp.arange(e_pad - e, dtype=jnp.int32) % 64
        src = jnp.concatenate([src, pad_i])
        dst = jnp.concatenate([dst, n + pad_i])

    accx, cnt, _unused_partials = _sc_segment_sum(src, dst, x2d, zx, n_pad)

    self_out = _tc_self(x2d, W_self.T.astype(jnp.float32),
                        b_self.reshape(1, d).astype(jnp.float32), n, d, 2000)

    degf = deg.astype(jnp.float32).reshape(n, 1)
    out2d = _tc_finish(
        self_out, accx, cnt.reshape(NC, n_pad, 1), degf,
        W_neigh.T.astype(jnp.float32),
        b_neigh.reshape(1, d).astype(jnp.float32),
        n, n_pad, d, rows=1000,
    )
    return out2d.reshape(b, n, d).astype(x.dtype)


# speedup vs baseline: 2.0524x; 1.0178x over previous
import dataclasses
import functools

import jax
import jax.numpy as jnp
from jax import lax
from jax.experimental import pallas as pl
from jax.experimental.pallas import tpu as pltpu
from jax.experimental.pallas import tpu_sc as plsc

NC = 2
NS = 16
NW = NC * NS
CH = 48


def _sc_segment_sum(src, dst, x2d, zx, n_pad):
    e = src.shape[0]
    d = x2d.shape[1]
    epw = e // NW
    cpw = epw // CH
    mesh = plsc.VectorSubcoreMesh(core_axis_name="c", subcore_axis_name="s", num_cores=NC)
    rows_per_sub = n_pad // NS

    cp = pltpu.CompilerParams()
    if "needs_layout_passes" in pltpu.CompilerParams.__dataclass_fields__:
        cp = dataclasses.replace(cp, needs_layout_passes=False)

    @functools.partial(
        pl.kernel,
        compiler_params=cp,
        out_type=[
            jax.ShapeDtypeStruct((NC, n_pad, d), jnp.float32),
            jax.ShapeDtypeStruct((NC * n_pad,), jnp.float32),
        ],
        mesh=mesh,
        scratch_types=(
            [pltpu.VMEM((CH,), jnp.int32)] * 4
            + [pltpu.VMEM((CH,), jnp.int32)] * 4
            + [pltpu.VMEM((CH, d), jnp.float32)] * 4
            + [
                pltpu.VMEM((n_pad,), jnp.float32),
                pltpu.VMEM((rows_per_sub,), jnp.float32),
                pltpu.VMEM((rows_per_sub,), jnp.float32),
                pltpu.VMEM_SHARED((n_pad, d), jnp.float32),
                pltpu.VMEM_SHARED((NS, n_pad), jnp.float32),
            ]
            + [pltpu.SemaphoreType.DMA] * 16
        ),
    )
    def seg_sum(src_hbm, dst_hbm, x_hbm, zx_hbm, accx_hbm, cnt_hbm,
                *refs):
        sidx_b = refs[0:4]
        didx_b = refs[4:8]
        rows_b = refs[8:12]
        cnt_tile, red_v, rin_v, accx_sh, cstage_sh = refs[12:17]
        gsem = refs[17:21]
        issem = refs[21:25]
        idsem = refs[25:29]
        ssem = refs[29:33]

        c = lax.axis_index("c")
        s = lax.axis_index("s")
        rbase = s * rows_per_sub
        wbase = (c * NS + s) * epw

        z16 = jnp.zeros((16,), jnp.float32)
        one16 = jnp.full((16,), 1.0, jnp.float32)

        def a_sidx(base, u):
            pltpu.make_async_copy(
                src_hbm.at[pl.ds(base, CH)], sidx_b[u], issem[u]).start()

        def a_didx(base, u):
            pltpu.make_async_copy(
                dst_hbm.at[pl.ds(base, CH)], didx_b[u], idsem[u]).start()

        def b_gather(base, u, wait_scatter):
            if wait_scatter:
                pltpu.make_async_copy(
                    rows_b[u], accx_sh.at[didx_b[u]], ssem[u]).wait()
            pltpu.make_async_copy(
                src_hbm.at[pl.ds(base, CH)], sidx_b[u], issem[u]).wait()
            pltpu.make_async_copy(
                x_hbm.at[sidx_b[u]], rows_b[u], gsem[u]).start()

        def c_consume(base, u):
            pltpu.make_async_copy(
                dst_hbm.at[pl.ds(base, CH)], didx_b[u], idsem[u]).wait()
            pltpu.make_async_copy(
                x_hbm.at[sidx_b[u]], rows_b[u], gsem[u]).wait()
            pltpu.make_async_copy(
                rows_b[u], accx_sh.at[didx_b[u]], ssem[u]).start(add=True)

            @pl.loop(0, CH, step=16)
            def _(j):
                idx16 = didx_b[u].at[pl.ds(j, 16)][...]
                plsc.addupdate_scatter(cnt_tile, [idx16], one16)

        for k in range(4):
            a_sidx(wbase + k * CH, k % 4)
        for k in range(3):
            a_didx(wbase + k * CH, k % 4)

        @pl.loop(0, n_pad, step=16)
        def _(r):
            cnt_tile.at[pl.ds(r, 16)][...] = z16

        pltpu.sync_copy(zx_hbm, accx_sh.at[pl.ds(rbase, rows_per_sub)])
        plsc.subcore_barrier()

        for k in range(3):
            b_gather(wbase + k * CH, k % 4, wait_scatter=False)

        def slot(k_dyn, u, wait_b):
            c_consume(k_dyn, u)
            a_sidx(k_dyn + 4 * CH, u)
            b_gather(k_dyn + 3 * CH, (u + 3) % 4, wait_scatter=wait_b)
            a_didx(k_dyn + 3 * CH, (u + 3) % 4)

        for u in range(4):
            slot(wbase + u * CH, u, wait_b=(u >= 1))

        @pl.loop(4, cpw - 4, step=4)
        def _(t):
            base = wbase + t * CH
            for u in range(4):
                slot(base + u * CH, u, wait_b=True)

        for u in range(4):
            k = cpw - 4 + u
            c_consume(wbase + k * CH, u)
            if k + 3 < cpw:
                b_gather(wbase + (k + 3) * CH, (u + 3) % 4, wait_scatter=True)
                a_didx(wbase + (k + 3) * CH, (u + 3) % 4)
        for u in range(4):
            pltpu.make_async_copy(
                rows_b[u], accx_sh.at[didx_b[u]], ssem[u]).wait()
        plsc.subcore_barrier()
        plsc.subcore_barrier()

        pltpu.sync_copy(accx_sh.at[pl.ds(rbase, rows_per_sub)],
                        accx_hbm.at[c, pl.ds(rbase, rows_per_sub)])

        pltpu.sync_copy(cnt_tile, cstage_sh.at[s])
        plsc.subcore_barrier()

        @pl.loop(0, rows_per_sub, step=16)
        def _(r):
            red_v.at[pl.ds(r, 16)][...] = z16

        for k in range(NS):
            pltpu.sync_copy(cstage_sh.at[k, pl.ds(rbase, rows_per_sub)], rin_v)

            @pl.loop(0, rows_per_sub, step=16)
            def _(r):
                red_v.at[pl.ds(r, 16)][...] = (
                    red_v.at[pl.ds(r, 16)][...] + rin_v.at[pl.ds(r, 16)][...])

        pltpu.sync_copy(red_v,
                        cnt_hbm.at[pl.ds(c * n_pad + rbase, rows_per_sub)])

    return seg_sum(src, dst, x2d, zx)


def _tc_self(x2d, wsT, bs, n, d, rows):
    def body(x_ref, wsT_ref, bs_ref, o_ref):
        o_ref[...] = jnp.dot(x_ref[...], wsT_ref[...],
                             preferred_element_type=jnp.float32) + bs_ref[...]

    return pl.pallas_call(
        body,
        grid=(n // rows,),
        in_specs=[
            pl.BlockSpec((rows, d), lambda i: (i, 0)),
            pl.BlockSpec((d, d), lambda i: (0, 0)),
            pl.BlockSpec((1, d), lambda i: (0, 0)),
        ],
        out_specs=pl.BlockSpec((rows, d), lambda i: (i, 0)),
        out_shape=jax.ShapeDtypeStruct((n, d), jnp.float32),
    )(x2d, wsT, bs)


def _tc_finish(self_out, accx, cnt3, degf, wnT, bn, n, n_pad, d, rows):
    grid = (n // rows,)

    def body(s_ref, a_ref, c_ref, deg_ref, wnT_ref, bn_ref, o_ref):
        acc = a_ref[0] + a_ref[1]
        cnt = c_ref[0] + c_ref[1]
        denom = jnp.maximum(deg_ref[...], 1.0)
        inv = 1.0 / denom
        neigh = jnp.dot(acc, wnT_ref[...], preferred_element_type=jnp.float32)
        out = s_ref[...] + (neigh + cnt * bn_ref[...]) * inv
        o_ref[...] = jnp.where(out >= 0, out, 0.1 * out)

    return pl.pallas_call(
        body,
        grid=grid,
        in_specs=[
            pl.BlockSpec((rows, d), lambda i: (i, 0)),
            pl.BlockSpec((NC, rows, d), lambda i: (0, i, 0)),
            pl.BlockSpec((NC, rows, 1), lambda i: (0, i, 0)),
            pl.BlockSpec((rows, 1), lambda i: (i, 0)),
            pl.BlockSpec((d, d), lambda i: (0, 0)),
            pl.BlockSpec((1, d), lambda i: (0, 0)),
        ],
        out_specs=pl.BlockSpec((rows, d), lambda i: (i, 0)),
        out_shape=jax.ShapeDtypeStruct((n, d), jnp.float32),
    )(self_out, accx, cnt3, degf, wnT, bn)


def kernel(x, edge_index, deg, W_self, b_self, W_neigh, b_neigh):
    b, n, d = x.shape
    e = edge_index.shape[1]
    assert b == 1

    src = edge_index[0].astype(jnp.int32)
    dst = edge_index[1].astype(jnp.int32)
    x2d = x.reshape(n, d).astype(jnp.float32)

    n_pad = ((n + 2047) // 2048) * 2048
    assert n_pad >= n + 64
    rows_per_sub = n_pad // NS
    zx = jnp.zeros((rows_per_sub, d), jnp.float32)

    quantum = NW * CH * 4
    e_pad = ((e + quantum - 1) // quantum) * quantum
    e_pad = max(e_pad, 2 * quantum)
    if e_pad != e:
        pad_i = jnp.arange(e_pad - e, dtype=jnp.int32) % 64
        src = jnp.concatenate([src, pad_i])
        dst = jnp.concatenate([dst, n + pad_i])

    accx, cnt = _sc_segment_sum(src, dst, x2d, zx, n_pad)

    self_out = _tc_self(x2d, W_self.T.astype(jnp.float32),
                        b_self.reshape(1, d).astype(jnp.float32), n, d, 2000)

    degf = deg.astype(jnp.float32).reshape(n, 1)
    out2d = _tc_finish(
        self_out, accx, cnt.reshape(NC, n_pad, 1), degf,
        W_neigh.T.astype(jnp.float32),
        b_neigh.reshape(1, d).astype(jnp.float32),
        n, n_pad, d, rows=1000,
    )
    return out2d.reshape(b, n, d).astype(x.dtype)
